# baseline (speedup 1.0000x reference)
import functools

import jax
import jax.numpy as jnp
from jax import lax
from jax.experimental import pallas as pl
from jax.experimental.pallas import tpu as pltpu
from jax.experimental.pallas import tpu_sc as plsc

D_MODEL = 64
D_PAD = 128
NUM_CORES = 2
NUM_SUBCORES = 16
NW = NUM_CORES * NUM_SUBCORES

CHUNK = 128
NBUF = 5
LA = 3


@functools.lru_cache(maxsize=None)
def _build(B):
    assert B % (NW * CHUNK * NBUF) == 0
    chunks_per_w = B // (NW * CHUNK)
    groups = chunks_per_w // NBUF
    mesh = plsc.VectorSubcoreMesh(core_axis_name="c", subcore_axis_name="s")

    @functools.partial(
        pl.kernel,
        mesh=mesh,
        out_type=jax.ShapeDtypeStruct((B, D_PAD), jnp.float32),
        scratch_types=[
            pltpu.VMEM((chunks_per_w, CHUNK), jnp.int32),
            pltpu.VMEM((NBUF, CHUNK, D_PAD), jnp.float32),
            pltpu.SemaphoreType.DMA((NBUF,)),
            pltpu.SemaphoreType.DMA((NBUF,)),
        ],
    )
    def emb(idx_hbm, table_hbm, out_hbm, idx_v, rows_v, gsem, osem):
        wid = lax.axis_index("s") * NUM_CORES + lax.axis_index("c")
        chunk0 = wid * chunks_per_w

        pltpu.sync_copy(idx_hbm.at[pl.ds(chunk0, chunks_per_w)], idx_v)

        def gather_start(c, b):
            pltpu.async_copy(table_hbm.at[idx_v.at[c]], rows_v.at[b],
                             gsem.at[b])

        def gather_wait(c, b):
            pltpu.make_async_copy(table_hbm.at[idx_v.at[c]], rows_v.at[b],
                                  gsem.at[b]).wait()

        def out_start(c, b):
            pltpu.async_copy(rows_v.at[b],
                             out_hbm.at[pl.ds((chunk0 + c) * CHUNK, CHUNK)],
                             osem.at[b])

        def out_wait(c, b):
            pltpu.make_async_copy(rows_v.at[b],
                                  out_hbm.at[pl.ds((chunk0 + c) * CHUNK,
                                                   CHUNK)],
                                  osem.at[b]).wait()

        nchunk = chunks_per_w

        for b in range(LA):
            gather_start(b, b)

        def body(g, carry):
            cbase = g * NBUF
            for b in range(NBUF):
                c = cbase + b
                gather_wait(c, b)
                out_start(c, b)
                c2 = c + LA
                b2 = (b + LA) % NBUF

                @pl.when(c2 < nchunk)
                def _issue():
                    @pl.when(c2 >= NBUF)
                    def _free():
                        out_wait(c2 - NBUF, b2)

                    gather_start(c2, b2)

            return carry

        lax.fori_loop(0, groups, body, 0, unroll=False)

        cbase = (groups - 1) * NBUF
        for b in range(NBUF):
            out_wait(cbase + b, b)

    return emb


def kernel(idx, E):
    nb, nt = idx.shape
    B = nb * nt
    Epad = jnp.pad(E, ((0, 0), (0, D_PAD - D_MODEL)))
    idx32 = idx.astype(jnp.int32).reshape(B // CHUNK, CHUNK)
    out_pad = _build(B)(idx32, Epad)
    return out_pad[:, :D_MODEL].reshape(nb, nt, D_MODEL)
